# baseline (speedup 1.0000x reference)
import functools

import jax
import jax.numpy as jnp
from jax.experimental import pallas as pl
from jax.experimental.pallas import tpu as pltpu

BETA = 0.25


def _vq_block(z_ref, cb_ref, zn_ref, cbn_ref, nz_ref, ne_ref,
              iota_ref, sim_ref, ids_ref, zq_ref, loss_ref, *,
              block_m, k_codes):
    z = z_ref[...]
    cb = cb_ref[...]

    dot = jax.lax.dot_general(
        z, cb, (((1,), (1,)), ((), ())),
        preferred_element_type=jnp.float32)

    ne_row = ne_ref[...]
    nz = nz_ref[...]
    dist = (-2.0 * dot + nz) + ne_row

    mind = jnp.min(dist, axis=1, keepdims=True)
    iotaf = iota_ref[...]
    candf = jnp.where(dist == mind, iotaf, float(k_codes))
    ids2f = jnp.min(candf, axis=1, keepdims=True)
    ids2 = ids2f.astype(jnp.int32)
    ids_ref[...] = ids2

    sim_ref[...] = jax.lax.dot_general(
        zn_ref[...], cbn_ref[...], (((1,), (1,)), ((), ())),
        preferred_element_type=jnp.float32)

    onehot = (iotaf == ids2f).astype(jnp.float32)
    zq = jax.lax.dot_general(
        onehot, cb, (((1,), (0,)), ((), ())),
        preferred_element_type=jnp.float32)
    zq_ref[...] = zq

    nrm = jnp.sqrt(jnp.maximum(mind, 0.0))
    loss_ref[...] = jnp.sum(nrm).reshape(1, 1, 1)


@jax.jit
def kernel(z_e, codebook):
    B, T, D = z_e.shape
    K = codebook.shape[0]
    N = B * T
    M = 1024
    num_blocks = N // M

    z_flat = z_e.reshape(N, D)
    nz = jnp.sum(jnp.square(z_flat), axis=1, keepdims=True)
    ne_col = jnp.sum(jnp.square(codebook), axis=1, keepdims=True)
    ne = ne_col.T
    z_n = z_flat / jnp.sqrt(nz)
    cb_n = codebook / jnp.sqrt(ne_col)
    iota_row = jnp.arange(K, dtype=jnp.float32)[None, :]
    body = functools.partial(_vq_block, block_m=M, k_codes=K)

    sim, ids2, zq, lacc = pl.pallas_call(
        body,
        grid=(num_blocks,),
        in_specs=[
            pl.BlockSpec((M, D), lambda i: (i, 0)),
            pl.BlockSpec((K, D), lambda i: (0, 0)),
            pl.BlockSpec((M, D), lambda i: (i, 0)),
            pl.BlockSpec((K, D), lambda i: (0, 0)),
            pl.BlockSpec((M, 1), lambda i: (i, 0)),
            pl.BlockSpec((1, K), lambda i: (0, 0)),
            pl.BlockSpec((1, K), lambda i: (0, 0)),
        ],
        out_specs=[
            pl.BlockSpec((M, K), lambda i: (i, 0)),
            pl.BlockSpec((M, 1), lambda i: (i, 0)),
            pl.BlockSpec((M, D), lambda i: (i, 0)),
            pl.BlockSpec((1, 1, 1), lambda i: (i, 0, 0)),
        ],
        out_shape=[
            jax.ShapeDtypeStruct((N, K), jnp.float32),
            jax.ShapeDtypeStruct((N, 1), jnp.int32),
            jax.ShapeDtypeStruct((N, D), jnp.float32),
            jax.ShapeDtypeStruct((num_blocks, 1, 1), jnp.float32),
        ],
        compiler_params=pltpu.CompilerParams(
            dimension_semantics=("arbitrary",)),
    )(z_flat, codebook, z_n, cb_n, nz, ne, iota_row)

    similarity = sim.reshape(B, T, K)
    ids = ids2.reshape(B, T)
    z_q = zq.reshape(B, T, D)
    loss = jnp.sum(lacc) * ((1.0 + BETA) / N)
    return z_q, similarity, ids, loss


# speedup vs baseline: 1.0849x; 1.0849x over previous
import functools

import jax
import jax.numpy as jnp
from jax.experimental import pallas as pl
from jax.experimental.pallas import tpu as pltpu

BETA = 0.25


def _vq_block(z_ref, cb_ref, nz_ref, ne_ref,
              rnz_ref, rne_ref, sim_ref, ids_ref, zq_ref, loss_ref, *,
              block_m, k_codes):
    z = z_ref[...]
    cb = cb_ref[...]

    dot = jax.lax.dot_general(
        z, cb, (((1,), (1,)), ((), ())),
        preferred_element_type=jnp.float32)

    ne_row = ne_ref[...]
    nz = nz_ref[...]
    dist = (-2.0 * dot + nz) + ne_row

    mind = jnp.min(dist, axis=1, keepdims=True)
    iotaf = jax.lax.broadcasted_iota(
        jnp.int32, (block_m, k_codes), 1).astype(jnp.float32)
    candf = jnp.where(dist == mind, iotaf, float(k_codes))
    ids2f = jnp.min(candf, axis=1, keepdims=True)
    ids2 = ids2f.astype(jnp.int32)
    ids_ref[...] = ids2

    scale = jax.lax.dot_general(
        rnz_ref[...], rne_ref[...], (((1,), (1,)), ((), ())),
        preferred_element_type=jnp.float32)
    sim_ref[...] = dot * scale

    onehot = (iotaf == ids2f).astype(jnp.float32)
    zq = jax.lax.dot_general(
        onehot, cb, (((1,), (0,)), ((), ())),
        preferred_element_type=jnp.float32)
    zq_ref[...] = zq

    nrm = jnp.sqrt(jnp.maximum(mind, 0.0))
    loss_ref[...] = jnp.sum(nrm).reshape(1, 1, 1)


@jax.jit
def kernel(z_e, codebook):
    B, T, D = z_e.shape
    K = codebook.shape[0]
    N = B * T
    M = 1152
    num_blocks = N // M

    z_flat = z_e.reshape(N, D)
    nz = jnp.sum(jnp.square(z_flat), axis=1, keepdims=True)
    ne_col = jnp.sum(jnp.square(codebook), axis=1, keepdims=True)
    ne = ne_col.T
    rnz = 1.0 / jnp.sqrt(nz)
    rne = 1.0 / jnp.sqrt(ne_col)
    body = functools.partial(_vq_block, block_m=M, k_codes=K)

    sim, ids2, zq, lacc = pl.pallas_call(
        body,
        grid=(num_blocks,),
        in_specs=[
            pl.BlockSpec((M, D), lambda i: (i, 0)),
            pl.BlockSpec((K, D), lambda i: (0, 0)),
            pl.BlockSpec((M, 1), lambda i: (i, 0)),
            pl.BlockSpec((1, K), lambda i: (0, 0)),
            pl.BlockSpec((M, 1), lambda i: (i, 0)),
            pl.BlockSpec((K, 1), lambda i: (0, 0)),
        ],
        out_specs=[
            pl.BlockSpec((M, K), lambda i: (i, 0)),
            pl.BlockSpec((M, 1), lambda i: (i, 0)),
            pl.BlockSpec((M, D), lambda i: (i, 0)),
            pl.BlockSpec((1, 1, 1), lambda i: (i, 0, 0)),
        ],
        out_shape=[
            jax.ShapeDtypeStruct((N, K), jnp.float32),
            jax.ShapeDtypeStruct((N, 1), jnp.int32),
            jax.ShapeDtypeStruct((N, D), jnp.float32),
            jax.ShapeDtypeStruct((num_blocks, 1, 1), jnp.float32),
        ],
        compiler_params=pltpu.CompilerParams(
            dimension_semantics=("arbitrary",)),
    )(z_flat, codebook, nz, ne, rnz, rne)

    similarity = sim.reshape(B, T, K)
    ids = ids2.reshape(B, T)
    z_q = zq.reshape(B, T, D)
    loss = jnp.sum(lacc) * ((1.0 + BETA) / N)
    return z_q, similarity, ids, loss
